# baseline (speedup 1.0000x reference)
import functools

import jax
import jax.numpy as jnp
from jax import lax
from jax.experimental import pallas as pl
from jax.experimental.pallas import tpu as pltpu
from jax.experimental.pallas import tpu_sc as plsc

N = 10000
E = 320000
C = 128
ALPHA = 0.15
NPROP = 5

NC = 2
NS = 16
NW = NC * NS
EPT = E // NW
K = 80
NCHUNK = EPT // K
NB = 5
BCH = NCHUNK // NB
NG = K // 16
NP = 10240
RPT = NP // NS
CB = C // 16


def _scale(rows, val_blk, k):
    def _group(g, c2):
        vv = val_blk[k, pl.ds(g * 16, 16)]
        for l in range(16):
            e = g * 16 + l
            b = vv[l]
            for c8 in range(CB):
                rows[e, pl.ds(c8 * 16, 16)] = rows[e, pl.ds(c8 * 16, 16)] * b
        return c2

    lax.fori_loop(0, NG, _group, 0)


def _spmm_body(x_hbm, col_hbm, row_hbm, val_hbm, out_hbm,
               acc, col_blk, row_blk, val_blk,
               rows0, rows1, rows2, g0, g1, g2, s0, s1, s2):
    cid = lax.axis_index("c")
    sid = lax.axis_index("s")
    w = cid * NS + sid

    z16 = jnp.zeros((16,), jnp.float32)

    def _zero_rows0(r, carry):
        for c8 in range(CB):
            rows0[r, pl.ds(c8 * 16, 16)] = z16
        return carry

    lax.fori_loop(0, K, _zero_rows0, 0)
    for j in range(RPT // K):
        pltpu.sync_copy(rows0, acc.at[pl.ds(sid * RPT + j * K, K)])

    plsc.subcore_barrier()

    bufs = ((rows0, g0, s0), (rows1, g1, s1), (rows2, g2, s2))

    def _wait_gather(buf):
        pltpu.make_async_copy(x_hbm.at[col_blk.at[0]], buf[0], buf[1]).wait()

    def _wait_scatter(buf):
        pltpu.make_async_copy(buf[0], acc.at[row_blk.at[0]], buf[2]).wait()

    def _phase(k, cur, nxt, issue_next, wait_nxt_scatter):
        if wait_nxt_scatter:
            _wait_scatter(nxt)
        if issue_next:
            pltpu.async_copy(x_hbm.at[col_blk.at[k + 1]], nxt[0], nxt[1])
        _wait_gather(cur)
        _scale(cur[0], val_blk, k)
        pltpu.async_copy(cur[0], acc.at[row_blk.at[k]], cur[2], add=True)

    def _block(blk, carry):
        pltpu.sync_copy(col_hbm.at[w, blk], col_blk)
        pltpu.sync_copy(row_hbm.at[w, blk], row_blk)
        pltpu.sync_copy(val_hbm.at[w, blk], val_blk)

        pltpu.async_copy(x_hbm.at[col_blk.at[0]], rows0, g0)
        _phase(0, bufs[0], bufs[1], True, False)
        _phase(1, bufs[1], bufs[2], True, False)

        def _triple(t, c2):
            k0 = 2 + 3 * t
            _phase(k0, bufs[2], bufs[0], True, True)
            _phase(k0 + 1, bufs[0], bufs[1], True, True)
            _phase(k0 + 2, bufs[1], bufs[2], True, True)
            return c2

        lax.fori_loop(0, (BCH - 4) // 3, _triple, 0)

        _phase(BCH - 2, bufs[2], bufs[0], True, True)
        _phase(BCH - 1, bufs[0], bufs[1], False, False)
        _wait_scatter(bufs[1])
        _wait_scatter(bufs[2])
        _wait_scatter(bufs[0])
        return carry

    lax.fori_loop(0, NB, _block, 0)

    plsc.subcore_barrier()
    pltpu.sync_copy(acc.at[pl.ds(sid * RPT, RPT)],
                    out_hbm.at[cid, pl.ds(sid * RPT, RPT)])


_spmm = functools.partial(
    pl.kernel,
    mesh=plsc.VectorSubcoreMesh(core_axis_name="c", subcore_axis_name="s"),
    out_type=jax.ShapeDtypeStruct((NC, NP, C), jnp.float32),
    scratch_types=[
        pltpu.VMEM_SHARED((NP, C), jnp.float32),
        pltpu.VMEM((BCH, K), jnp.int32),
        pltpu.VMEM((BCH, K), jnp.int32),
        pltpu.VMEM((BCH, K), jnp.float32),
        pltpu.VMEM((K, C), jnp.float32),
        pltpu.VMEM((K, C), jnp.float32),
        pltpu.VMEM((K, C), jnp.float32),
        pltpu.SemaphoreType.DMA,
        pltpu.SemaphoreType.DMA,
        pltpu.SemaphoreType.DMA,
        pltpu.SemaphoreType.DMA,
        pltpu.SemaphoreType.DMA,
        pltpu.SemaphoreType.DMA,
    ],
)(_spmm_body)


_CBLK = 1000


def _combine_body(init_ref, p_ref, out_ref):
    out_ref[...] = (ALPHA * init_ref[...]
                    + (1.0 - ALPHA) * (p_ref[0] + p_ref[1]))


_combine = pl.pallas_call(
    _combine_body,
    out_shape=jax.ShapeDtypeStruct((N, C), jnp.float32),
    grid=(N // _CBLK,),
    in_specs=[
        pl.BlockSpec((_CBLK, C), lambda i: (i, 0)),
        pl.BlockSpec((NC, _CBLK, C), lambda i: (0, i, 0)),
    ],
    out_specs=pl.BlockSpec((_CBLK, C), lambda i: (i, 0)),
)


def kernel(logits, A_hat_indices, A_hat_values):
    row3 = A_hat_indices[0].reshape(NW, NB, BCH, K)
    col3 = A_hat_indices[1].reshape(NW, NB, BCH, K)
    val3 = A_hat_values.reshape(NW, NB, BCH, K)
    x = logits
    for _ in range(NPROP):
        partials = _spmm(x, col3, row3, val3)
        x = _combine(logits, partials)
    return x


# speedup vs baseline: 1.0558x; 1.0558x over previous
import functools

import jax
import jax.numpy as jnp
from jax import lax
from jax.experimental import pallas as pl
from jax.experimental.pallas import tpu as pltpu
from jax.experimental.pallas import tpu_sc as plsc

N = 10000
E = 320000
C = 128
ALPHA = 0.15
NPROP = 5

NC = 2
NS = 16
NW = NC * NS
EPT = E // NW
K = 80
NCHUNK = EPT // K
NB = 5
BCH = NCHUNK // NB
NG = K // 16
NP = 10240
RPT = NP // NS
CB = C // 16


def _scale(rows, val_blk, k):
    def _group(g, c2):
        vv = val_blk[k, pl.ds(g * 16, 16)]
        for l in range(16):
            e = g * 16 + l
            b = vv.at[jnp.full((16,), l, jnp.int32)].get(
                mode="promise_in_bounds")
            for c8 in range(CB):
                rows[e, pl.ds(c8 * 16, 16)] = rows[e, pl.ds(c8 * 16, 16)] * b
        return c2

    lax.fori_loop(0, NG, _group, 0)


def _spmm_body(x_hbm, col_hbm, row_hbm, val_hbm, out_hbm,
               acc, col_blk, row_blk, val_blk,
               rows0, rows1, rows2, g0, g1, g2, s0, s1, s2, si):
    cid = lax.axis_index("c")
    sid = lax.axis_index("s")
    w = cid * NS + sid

    z16 = jnp.zeros((16,), jnp.float32)

    def _zero_rows0(r, carry):
        for c8 in range(CB):
            rows0[r, pl.ds(c8 * 16, 16)] = z16
        return carry

    lax.fori_loop(0, K, _zero_rows0, 0)
    for j in range(RPT // K):
        pltpu.sync_copy(rows0, acc.at[pl.ds(sid * RPT + j * K, K)])

    plsc.subcore_barrier()

    bufs = ((rows0, g0, s0), (rows1, g1, s1), (rows2, g2, s2))

    def _wait_gather(buf):
        pltpu.make_async_copy(x_hbm.at[col_blk.at[0]], buf[0], buf[1]).wait()

    def _wait_scatter(buf):
        pltpu.make_async_copy(buf[0], acc.at[row_blk.at[0]], buf[2]).wait()

    def _phase(k, cur, nxt, issue_next, wait_nxt_scatter):
        if wait_nxt_scatter:
            _wait_scatter(nxt)
        if issue_next:
            pltpu.async_copy(x_hbm.at[col_blk.at[k + 1]], nxt[0], nxt[1])
        _wait_gather(cur)
        _scale(cur[0], val_blk, k)
        pltpu.async_copy(cur[0], acc.at[row_blk.at[k]], cur[2], add=True)

    def _block(blk, carry):
        pltpu.async_copy(col_hbm.at[w, blk], col_blk, si)
        pltpu.async_copy(row_hbm.at[w, blk], row_blk, si)
        pltpu.async_copy(val_hbm.at[w, blk], val_blk, si)
        pltpu.make_async_copy(col_hbm.at[w, 0], col_blk, si).wait()
        pltpu.make_async_copy(row_hbm.at[w, 0], row_blk, si).wait()
        pltpu.make_async_copy(val_hbm.at[w, 0], val_blk, si).wait()

        pltpu.async_copy(x_hbm.at[col_blk.at[0]], rows0, g0)
        _phase(0, bufs[0], bufs[1], True, False)
        _phase(1, bufs[1], bufs[2], True, False)

        def _triple(t, c2):
            k0 = 2 + 3 * t
            _phase(k0, bufs[2], bufs[0], True, True)
            _phase(k0 + 1, bufs[0], bufs[1], True, True)
            _phase(k0 + 2, bufs[1], bufs[2], True, True)
            return c2

        lax.fori_loop(0, (BCH - 4) // 3, _triple, 0)

        _phase(BCH - 2, bufs[2], bufs[0], True, True)
        _phase(BCH - 1, bufs[0], bufs[1], False, False)
        _wait_scatter(bufs[1])
        _wait_scatter(bufs[2])
        _wait_scatter(bufs[0])
        return carry

    lax.fori_loop(0, NB, _block, 0)

    plsc.subcore_barrier()
    pltpu.sync_copy(acc.at[pl.ds(sid * RPT, RPT)],
                    out_hbm.at[cid, pl.ds(sid * RPT, RPT)])


_spmm = functools.partial(
    pl.kernel,
    mesh=plsc.VectorSubcoreMesh(core_axis_name="c", subcore_axis_name="s"),
    out_type=jax.ShapeDtypeStruct((NC, NP, C), jnp.float32),
    scratch_types=[
        pltpu.VMEM_SHARED((NP, C), jnp.float32),
        pltpu.VMEM((BCH, K), jnp.int32),
        pltpu.VMEM((BCH, K), jnp.int32),
        pltpu.VMEM((BCH, K), jnp.float32),
        pltpu.VMEM((K, C), jnp.float32),
        pltpu.VMEM((K, C), jnp.float32),
        pltpu.VMEM((K, C), jnp.float32),
        pltpu.SemaphoreType.DMA,
        pltpu.SemaphoreType.DMA,
        pltpu.SemaphoreType.DMA,
        pltpu.SemaphoreType.DMA,
        pltpu.SemaphoreType.DMA,
        pltpu.SemaphoreType.DMA,
        pltpu.SemaphoreType.DMA,
    ],
)(_spmm_body)


_CBLK = 1000


def _combine_body(init_ref, p_ref, out_ref):
    out_ref[...] = (ALPHA * init_ref[...]
                    + (1.0 - ALPHA) * (p_ref[0] + p_ref[1]))


_combine = pl.pallas_call(
    _combine_body,
    out_shape=jax.ShapeDtypeStruct((N, C), jnp.float32),
    grid=(N // _CBLK,),
    in_specs=[
        pl.BlockSpec((_CBLK, C), lambda i: (i, 0)),
        pl.BlockSpec((NC, _CBLK, C), lambda i: (0, i, 0)),
    ],
    out_specs=pl.BlockSpec((_CBLK, C), lambda i: (i, 0)),
)


def kernel(logits, A_hat_indices, A_hat_values):
    row3 = A_hat_indices[0].reshape(NW, NB, BCH, K)
    col3 = A_hat_indices[1].reshape(NW, NB, BCH, K)
    val3 = A_hat_values.reshape(NW, NB, BCH, K)
    x = logits
    for _ in range(NPROP):
        partials = _spmm(x, col3, row3, val3)
        x = _combine(logits, partials)
    return x
